# baseline (speedup 1.0000x reference)
import functools

import jax
import jax.numpy as jnp
from jax import lax
from jax.experimental import pallas as pl
from jax.experimental.pallas import tpu as pltpu
from jax.experimental.pallas import tpu_sc as plsc

N_CAT = 32
N_INT = 32
KEY_EMBED = 128
VAL_EMBED = 128
EMBED = KEY_EMBED + VAL_EMBED
L = 16


def _body(cat_idx_hbm, int_values_hbm, cat_flat_hbm, int_key_hbm, w_hbm,
          b_hbm, out_hbm, idx_v, buf_v, key_v, w_v, b_v, val_v, sem):
    wid = lax.axis_index("s") * 2 + lax.axis_index("c")

    @pl.when(wid < 2)
    def _cat():
        base = wid * L
        pltpu.sync_copy(cat_idx_hbm.at[pl.ds(base, L)], idx_v)
        ci = idx_v[...]
        row = jax.lax.iota(jnp.int32, L) + base
        idx_v[...] = row * 3 + ci
        pltpu.async_copy(cat_flat_hbm.at[idx_v], buf_v, sem).wait()
        pltpu.sync_copy(buf_v, out_hbm.at[pl.ds(base, L)])

    @pl.when(jnp.logical_and(wid >= 2, wid < 4))
    def _int():
        j0 = (wid - 2) * L
        pltpu.sync_copy(int_key_hbm.at[pl.ds(j0, L)], key_v)
        pltpu.sync_copy(w_hbm.at[pl.ds(j0, L)], w_v)
        pltpu.sync_copy(b_hbm.at[pl.ds(j0, L)], b_v)
        pltpu.sync_copy(int_values_hbm.at[pl.ds(j0, L)], val_v)
        vals = val_v[...]
        for r in range(L):
            vvec = vals[jnp.full((L,), r, jnp.int32)]
            for c in range(KEY_EMBED // L):
                sl = pl.ds(c * L, L)
                buf_v[r, sl] = key_v[r, sl]
                buf_v[r, pl.ds(KEY_EMBED + c * L, L)] = (
                    vvec * w_v[r, sl] + b_v[r, sl])
        pltpu.sync_copy(buf_v, out_hbm.at[pl.ds(N_CAT + j0, L)])


@functools.partial(
    pl.kernel,
    out_type=jax.ShapeDtypeStruct((N_CAT + N_INT, EMBED), jnp.float32),
    mesh=plsc.VectorSubcoreMesh(core_axis_name="c", subcore_axis_name="s"),
    scratch_types=[
        pltpu.VMEM((L,), jnp.int32),
        pltpu.VMEM((L, EMBED), jnp.float32),
        pltpu.VMEM((L, KEY_EMBED), jnp.float32),
        pltpu.VMEM((L, VAL_EMBED), jnp.float32),
        pltpu.VMEM((L, VAL_EMBED), jnp.float32),
        pltpu.VMEM((L,), jnp.float32),
        pltpu.SemaphoreType.DMA,
    ],
)
def _sc_embed(*refs):
    _body(*refs)


def kernel(cat_indices, int_values, cat_table, int_key_table, W, b):
    cat_flat = cat_table.reshape(N_CAT * 3, EMBED)
    return _sc_embed(cat_indices.astype(jnp.int32), int_values, cat_flat,
                     int_key_table, W, b)


# speedup vs baseline: 1.0703x; 1.0703x over previous
import functools

import jax
import jax.numpy as jnp
from jax import lax
from jax.experimental import pallas as pl
from jax.experimental.pallas import tpu as pltpu
from jax.experimental.pallas import tpu_sc as plsc

N_CAT = 32
N_INT = 32
KEY_EMBED = 128
VAL_EMBED = 128
EMBED = KEY_EMBED + VAL_EMBED
L = 16


def _body(cat_idx_hbm, int_values_hbm, cat_flat_hbm, int_key_hbm, w_hbm,
          b_hbm, out_hbm, idx_v, buf_v, key_v, w_v, b_v, val_v, sem):
    wid = lax.axis_index("s")

    @pl.when(wid < 2)
    def _cat():
        base = wid * L
        pltpu.sync_copy(cat_idx_hbm.at[pl.ds(base, L)], idx_v)
        ci = idx_v[...]
        row = jax.lax.iota(jnp.int32, L) + base
        idx_v[...] = row * 3 + ci
        pltpu.async_copy(cat_flat_hbm.at[idx_v], buf_v, sem).wait()
        pltpu.sync_copy(buf_v, out_hbm.at[pl.ds(base, L)])

    @pl.when(jnp.logical_and(wid >= 2, wid < 4))
    def _int():
        j0 = (wid - 2) * L
        pltpu.sync_copy(int_key_hbm.at[pl.ds(j0, L)], key_v)
        pltpu.sync_copy(w_hbm.at[pl.ds(j0, L)], w_v)
        pltpu.sync_copy(b_hbm.at[pl.ds(j0, L)], b_v)
        pltpu.sync_copy(int_values_hbm.at[pl.ds(j0, L)], val_v)
        vals = val_v[...]
        for r in range(L):
            vvec = vals[jnp.full((L,), r, jnp.int32)]
            for c in range(KEY_EMBED // L):
                sl = pl.ds(c * L, L)
                buf_v[r, sl] = key_v[r, sl]
                buf_v[r, pl.ds(KEY_EMBED + c * L, L)] = (
                    vvec * w_v[r, sl] + b_v[r, sl])
        pltpu.sync_copy(buf_v, out_hbm.at[pl.ds(N_CAT + j0, L)])


@functools.partial(
    pl.kernel,
    out_type=jax.ShapeDtypeStruct((N_CAT + N_INT, EMBED), jnp.float32),
    mesh=plsc.VectorSubcoreMesh(core_axis_name="c", subcore_axis_name="s",
                                num_cores=1),
    scratch_types=[
        pltpu.VMEM((L,), jnp.int32),
        pltpu.VMEM((L, EMBED), jnp.float32),
        pltpu.VMEM((L, KEY_EMBED), jnp.float32),
        pltpu.VMEM((L, VAL_EMBED), jnp.float32),
        pltpu.VMEM((L, VAL_EMBED), jnp.float32),
        pltpu.VMEM((L,), jnp.float32),
        pltpu.SemaphoreType.DMA,
    ],
)
def _sc_embed(*refs):
    _body(*refs)


def kernel(cat_indices, int_values, cat_table, int_key_table, W, b):
    cat_flat = cat_table.reshape(N_CAT * 3, EMBED)
    return _sc_embed(cat_indices.astype(jnp.int32), int_values, cat_flat,
                     int_key_table, W, b)


# speedup vs baseline: 1.1491x; 1.0736x over previous
import functools

import jax
import jax.numpy as jnp
from jax import lax
from jax.experimental import pallas as pl
from jax.experimental.pallas import tpu as pltpu
from jax.experimental.pallas import tpu_sc as plsc

N_CAT = 32
N_INT = 32
KEY_EMBED = 128
VAL_EMBED = 128
EMBED = KEY_EMBED + VAL_EMBED
L = 16


def _body(cat_idx_hbm, int_values_hbm, cat_flat_hbm, int_key_hbm, w_hbm,
          b_hbm, out_hbm, idx_v, gbuf_v, vbuf_v, w_v, b_v, val_v, sem):
    wid = lax.axis_index("s")

    @pl.when(wid < 2)
    def _cat():
        base = wid * L
        pltpu.sync_copy(cat_idx_hbm.at[pl.ds(base, L)], idx_v)
        ci = idx_v[...]
        row = jax.lax.iota(jnp.int32, L) + base
        idx_v[...] = row * 3 + ci
        pltpu.async_copy(cat_flat_hbm.at[idx_v], gbuf_v, sem).wait()
        pltpu.sync_copy(gbuf_v, out_hbm.at[pl.ds(base, L)])

    @pl.when(jnp.logical_and(wid >= 2, wid < 4))
    def _int():
        j0 = (wid - 2) * L
        cw = pltpu.async_copy(w_hbm.at[pl.ds(j0, L)], w_v, sem)
        cb = pltpu.async_copy(b_hbm.at[pl.ds(j0, L)], b_v, sem)
        cv = pltpu.async_copy(int_values_hbm.at[pl.ds(j0, L)], val_v, sem)
        cw.wait()
        cb.wait()
        cv.wait()
        vals = val_v[...]
        for r in range(L):
            vvec = vals[jnp.full((L,), r, jnp.int32)]
            for c in range(VAL_EMBED // L):
                sl = pl.ds(c * L, L)
                vbuf_v[r, sl] = vvec * w_v[r, sl] + b_v[r, sl]
        pltpu.sync_copy(
            vbuf_v, out_hbm.at[pl.ds(N_CAT + j0, L), pl.ds(KEY_EMBED, VAL_EMBED)])

    @pl.when(wid == 4)
    def _key():
        pltpu.sync_copy(
            int_key_hbm, out_hbm.at[pl.ds(N_CAT, N_INT), pl.ds(0, KEY_EMBED)])


@functools.partial(
    pl.kernel,
    out_type=jax.ShapeDtypeStruct((N_CAT + N_INT, EMBED), jnp.float32),
    mesh=plsc.VectorSubcoreMesh(core_axis_name="c", subcore_axis_name="s",
                                num_cores=1),
    scratch_types=[
        pltpu.VMEM((L,), jnp.int32),
        pltpu.VMEM((L, EMBED), jnp.float32),
        pltpu.VMEM((L, VAL_EMBED), jnp.float32),
        pltpu.VMEM((L, VAL_EMBED), jnp.float32),
        pltpu.VMEM((L, VAL_EMBED), jnp.float32),
        pltpu.VMEM((L,), jnp.float32),
        pltpu.SemaphoreType.DMA,
    ],
)
def _sc_embed(*refs):
    _body(*refs)


def kernel(cat_indices, int_values, cat_table, int_key_table, W, b):
    cat_flat = cat_table.reshape(N_CAT * 3, EMBED)
    return _sc_embed(cat_indices.astype(jnp.int32), int_values, cat_flat,
                     int_key_table, W, b)


# speedup vs baseline: 1.1905x; 1.0361x over previous
import functools

import jax
import jax.numpy as jnp
from jax import lax
from jax.experimental import pallas as pl
from jax.experimental.pallas import tpu as pltpu
from jax.experimental.pallas import tpu_sc as plsc

N_CAT = 32
N_INT = 32
KEY_EMBED = 128
VAL_EMBED = 128
EMBED = KEY_EMBED + VAL_EMBED
L = 16


def _body(cat_idx_hbm, int_values_hbm, cat_flat_hbm, int_key_hbm, w_hbm,
          b_hbm, out_hbm, idx_v, gbuf_v, vbuf_v, w_v, b_v, val_v, sem):
    wid = lax.axis_index("s")

    @pl.when(wid < 2)
    def _cat():
        base = wid * L
        pltpu.sync_copy(cat_idx_hbm.at[pl.ds(base, L)], idx_v)
        ci = idx_v[...]
        row = jax.lax.iota(jnp.int32, L) + base
        idx_v[...] = row * 3 + ci
        pltpu.async_copy(cat_flat_hbm.at[idx_v], gbuf_v, sem).wait()
        pltpu.sync_copy(gbuf_v, out_hbm.at[pl.ds(base, L)])

    R = 8

    @pl.when(jnp.logical_and(wid >= 2, wid < 6))
    def _int():
        j0 = (wid - 2) * R
        cw = pltpu.async_copy(w_hbm.at[pl.ds(j0, R)], w_v, sem)
        cb = pltpu.async_copy(b_hbm.at[pl.ds(j0, R)], b_v, sem)
        cv = pltpu.async_copy(int_values_hbm.at[pl.ds(j0, R)],
                              val_v.at[pl.ds(0, R)], sem)
        cw.wait()
        cb.wait()
        cv.wait()
        vals = val_v[...]
        for r in range(R):
            vvec = vals[jnp.full((L,), r, jnp.int32)]
            for c in range(VAL_EMBED // L):
                sl = pl.ds(c * L, L)
                vbuf_v[r, sl] = vvec * w_v[r, sl] + b_v[r, sl]
        pltpu.sync_copy(
            vbuf_v, out_hbm.at[pl.ds(N_CAT + j0, R), pl.ds(KEY_EMBED, VAL_EMBED)])

    @pl.when(wid == 6)
    def _key():
        pltpu.sync_copy(
            int_key_hbm, out_hbm.at[pl.ds(N_CAT, N_INT), pl.ds(0, KEY_EMBED)])


@functools.partial(
    pl.kernel,
    out_type=jax.ShapeDtypeStruct((N_CAT + N_INT, EMBED), jnp.float32),
    mesh=plsc.VectorSubcoreMesh(core_axis_name="c", subcore_axis_name="s",
                                num_cores=1),
    scratch_types=[
        pltpu.VMEM((L,), jnp.int32),
        pltpu.VMEM((L, EMBED), jnp.float32),
        pltpu.VMEM((8, VAL_EMBED), jnp.float32),
        pltpu.VMEM((8, VAL_EMBED), jnp.float32),
        pltpu.VMEM((8, VAL_EMBED), jnp.float32),
        pltpu.VMEM((L,), jnp.float32),
        pltpu.SemaphoreType.DMA,
    ],
)
def _sc_embed(*refs):
    _body(*refs)


def kernel(cat_indices, int_values, cat_table, int_key_table, W, b):
    cat_flat = cat_table.reshape(N_CAT * 3, EMBED)
    return _sc_embed(cat_indices.astype(jnp.int32), int_values, cat_flat,
                     int_key_table, W, b)
